# baseline (speedup 1.0000x reference)
import jax
import jax.numpy as jnp
from jax.experimental import pallas as pl
from jax.experimental.pallas import tpu as pltpu

_C = 256
_NH = 16
_MM_DTYPE = jnp.bfloat16
_N = 2
_LEVELS = ((128, 2048), (64, 2048), (32, 1024), (16, 256))


def _jobs():
    out = []
    for li, (S, BM) in enumerate(_LEVELS):
        SS = S * S
        for n in range(_N):
            for b in range(SS // BM):
                m0 = b * BM
                r0, r1 = m0 // S, (m0 + BM) // S
                rlo = max(0, r0 - 2) // 8 * 8
                rhi = min(S, (min(S, r1 + 2) + 7) // 8 * 8)
                zhead = (S + 8 - m0) if rlo == 0 and m0 - S - 8 < 0 else 0
                out.append((li, n, S, BM, m0, rlo, rhi, zhead))
    return out


_RBMAX = [max(j[6] - j[5] for j in _jobs() if j[0] == li)
          for li in range(len(_LEVELS))]
_XTR = max(max(j[7] + (j[6] - j[5]) * j[2],
               j[4] + j[3] + j[2] + 8 - (j[5] * j[2] - j[7]))
           for j in _jobs())


def _rpn_body(x3_ref, x4_ref, x5_ref, x6_ref, w_ref, cb_ref, hw_ref, hb_ref,
              oc3, ob3, oc4, ob4, oc5, ob5, oc6, ob6,
              st3, st4, st5, st6, xt_ref, sem_ref):
    x_refs = (x3_ref, x4_ref, x5_ref, x6_ref)
    st_refs = (st3, st4, st5, st6)
    oc_refs = (oc3, oc4, oc5, oc6)
    ob_refs = (ob3, ob4, ob5, ob6)
    jobs = _jobs()

    def copy(i):
        li, n, S, BM, m0, rlo, rhi, zhead = jobs[i]
        return pltpu.make_async_copy(
            x_refs[li].at[n, :, rlo:rhi, :],
            st_refs[li].at[i % 2, :, 0:rhi - rlo, :],
            sem_ref.at[i % 2])

    copy(0).start()
    for i, (li, n, S, BM, m0, rlo, rhi, zhead) in enumerate(jobs):
        if i + 1 < len(jobs):
            copy(i + 1).start()
        buf = i % 2
        RB = rhi - rlo
        copy(i).wait()

        poff = rlo * S - zhead
        rows = m0 + BM + S + 8 - poff
        if zhead:
            xt_ref[0:zhead, :] = jnp.zeros((zhead, _C), _MM_DTYPE)
        if zhead + RB * S < rows:
            xt_ref[zhead + RB * S:rows, :] = jnp.zeros(
                (rows - zhead - RB * S, _C), _MM_DTYPE)
        if S == 128:
            v = st_refs[li][buf, :, 0:RB, :].reshape(_C, RB * S)
            xt_ref[zhead:zhead + RB * S, :] = (
                jnp.transpose(v).astype(_MM_DTYPE))
        else:
            for r in range(RB):
                xt_ref[zhead + r * S:zhead + (r + 1) * S, :] = jnp.transpose(
                    st_refs[li][buf, :, r, :]).astype(_MM_DTYPE)

        xs3 = jnp.concatenate(
            [xt_ref[pl.ds(m0 + (dy - 1) * S - 8 - poff, BM + 16), :]
             for dy in range(3)],
            axis=1)
        accs = [jnp.dot(xs3, w_ref[dx], preferred_element_type=jnp.float32)
                for dx in range(3)]
        col = (jax.lax.broadcasted_iota(jnp.int32, (BM, 1), 0) + m0) & (S - 1)
        a0 = jnp.where(col != 0, accs[0][7:BM + 7], 0.0)
        a2 = jnp.where(col != S - 1, accs[2][9:BM + 9], 0.0)
        conv = a0 + accs[1][8:BM + 8] + a2
        t = jnp.maximum(conv + cb_ref[0, :][None, :], 0.0)
        h = jnp.dot(t.astype(_MM_DTYPE), hw_ref[...],
                    preferred_element_type=jnp.float32) + hb_ref[0, :][None, :]
        ht = jnp.transpose(h)
        oc_refs[li][n, :, m0:m0 + BM] = ht[:3]
        ob_refs[li][n, :, m0:m0 + BM] = ht[3:15]


@jax.jit
def _rpn_all(x3, x4, x5, x6, w3, cb, hw, hb):
    out_shape = []
    for S, _ in _LEVELS:
        out_shape += [jax.ShapeDtypeStruct((_N, 3, S * S), jnp.float32),
                      jax.ShapeDtypeStruct((_N, 12, S * S), jnp.float32)]

    any_spec = pl.BlockSpec(memory_space=pl.ANY)
    o = pl.pallas_call(
        _rpn_body,
        in_specs=[any_spec] * 4 + [
            pl.BlockSpec((3, 3 * _C, _C), lambda: (0, 0, 0)),
            pl.BlockSpec((1, _C), lambda: (0, 0)),
            pl.BlockSpec((_C, _NH), lambda: (0, 0)),
            pl.BlockSpec((1, _NH), lambda: (0, 0)),
        ],
        out_shape=out_shape,
        scratch_shapes=[
            pltpu.VMEM((2, _C, _RBMAX[0], 128), jnp.float32),
            pltpu.VMEM((2, _C, _RBMAX[1], 64), jnp.float32),
            pltpu.VMEM((2, _C, _RBMAX[2], 32), jnp.float32),
            pltpu.VMEM((2, _C, _RBMAX[3], 16), jnp.float32),
            pltpu.VMEM((_XTR, _C), _MM_DTYPE),
            pltpu.SemaphoreType.DMA((2,)),
        ],
    )(x3, x4, x5, x6, w3, cb, hw, hb)

    cls_out, bbox_out = [], []
    for li, (S, _) in enumerate(_LEVELS):
        cls_out.append(o[2 * li].reshape(_N, 3, S, S))
        bbox_out.append(o[2 * li + 1].reshape(_N, 12, S, S))
    return tuple(cls_out) + tuple(bbox_out)


def kernel(feat_p3, feat_p4, feat_p5, feat_p6,
           conv_w, conv_b, cls_w, cls_b, bbox_w, bbox_b):
    w3 = jnp.transpose(conv_w, (3, 2, 1, 0)).reshape(3, 3 * _C, _C)
    w3 = w3.astype(_MM_DTYPE)
    cb = conv_b.reshape(1, _C)
    hw = jnp.concatenate([cls_w[:, :, 0, 0], bbox_w[:, :, 0, 0]], axis=0)
    hw = jnp.pad(hw, ((0, 1), (0, 0))).T.astype(_MM_DTYPE)
    hb = jnp.pad(jnp.concatenate([cls_b, bbox_b]), (0, 1)).reshape(1, _NH)

    return _rpn_all(feat_p3, feat_p4, feat_p5, feat_p6, w3, cb, hw, hb)


# speedup vs baseline: 1.4671x; 1.4671x over previous
import jax
import jax.numpy as jnp
from jax.experimental import pallas as pl
from jax.experimental.pallas import tpu as pltpu

_C = 256
_NH = 16
_MM_DTYPE = jnp.bfloat16
_N = 2
_LEVELS = ((128, 2048), (64, 2048), (32, 1024), (16, 256))


def _jobs():
    out = []
    for li, (S, BM) in enumerate(_LEVELS):
        SS = S * S
        for n in range(_N):
            for b in range(SS // BM):
                m0 = b * BM
                if li == 0:
                    r0, r1 = m0 // S, (m0 + BM) // S
                    rlo = max(0, r0 - 2) // 8 * 8
                    rhi = min(S, (min(S, r1 + 2) + 7) // 8 * 8)
                    zhead = (S + 8 - m0) if rlo == 0 and m0 < S + 8 else 0
                    out.append((li, n, S, BM, m0, rlo, rhi, zhead))
                else:
                    lo = max(0, m0 - S - 8)
                    hi = min(SS, m0 + BM + S + 8)
                    zhead = lo - (m0 - S - 8)
                    out.append((li, n, S, BM, m0, lo, hi, zhead))
    return out


_RB3 = max(j[6] - j[5] for j in _jobs() if j[0] == 0)
_XTR = max(max(j[7] + (j[6] - j[5]) * j[2],
               j[4] + j[3] + j[2] + 8 - (j[5] * j[2] - j[7]))
           for j in _jobs() if j[0] == 0)


def _rpn_body(x3_ref, x4_ref, x5_ref, x6_ref, w_ref, cb_ref, hw_ref, hb_ref,
              oc3, ob3, oc4, ob4, oc5, ob5, oc6, ob6,
              st3, st4, st5, st6, xt_ref, sem_ref):
    x_refs = (x3_ref, x4_ref, x5_ref, x6_ref)
    st_refs = (st3, st4, st5, st6)
    oc_refs = (oc3, oc4, oc5, oc6)
    ob_refs = (ob3, ob4, ob5, ob6)
    jobs = _jobs()

    def copy(i):
        li, n, S, BM, m0, lo, hi, zhead = jobs[i]
        if li == 0:
            return pltpu.make_async_copy(
                x_refs[0].at[n, :, lo:hi, :],
                st_refs[0].at[i % 2, :, 0:hi - lo, :],
                sem_ref.at[i % 2])
        return pltpu.make_async_copy(
            x_refs[li].at[n, lo:hi, :],
            st_refs[li].at[i % 2, zhead:zhead + hi - lo, :],
            sem_ref.at[i % 2])

    copy(0).start()
    for i, (li, n, S, BM, m0, lo, hi, zhead) in enumerate(jobs):
        if i + 1 < len(jobs):
            copy(i + 1).start()
        buf = i % 2
        copy(i).wait()

        if li == 0:
            RB = hi - lo
            poff = lo * S - zhead
            rows = m0 + BM + S + 8 - poff
            if zhead:
                xt_ref[0:zhead, :] = jnp.zeros((zhead, _C), _MM_DTYPE)
            if zhead + RB * S < rows:
                xt_ref[zhead + RB * S:rows, :] = jnp.zeros(
                    (rows - zhead - RB * S, _C), _MM_DTYPE)
            v = st_refs[0][buf, :, 0:RB, :].reshape(_C, RB * S)
            xt_ref[zhead:zhead + RB * S, :] = (
                jnp.transpose(v).astype(_MM_DTYPE))
            base = m0 - S - 8 - poff
            xs3 = jnp.concatenate(
                [xt_ref[pl.ds(base + dy * S, BM + 16), :] for dy in range(3)],
                axis=1)
        else:
            rows = BM + 2 * S + 16
            if zhead:
                st_refs[li][buf, 0:zhead, :] = jnp.zeros(
                    (zhead, _C), jnp.float32)
            if zhead + hi - lo < rows:
                st_refs[li][buf, zhead + hi - lo:rows, :] = jnp.zeros(
                    (rows - zhead - hi + lo, _C), jnp.float32)
            xs3 = jnp.concatenate(
                [st_refs[li][buf, pl.ds(dy * S, BM + 16), :]
                 for dy in range(3)],
                axis=1).astype(_MM_DTYPE)

        accs = [jnp.dot(xs3, w_ref[dx], preferred_element_type=jnp.float32)
                for dx in range(3)]
        col = (jax.lax.broadcasted_iota(jnp.int32, (BM, 1), 0) + m0) & (S - 1)
        a0 = jnp.where(col != 0, accs[0][7:BM + 7], 0.0)
        a2 = jnp.where(col != S - 1, accs[2][9:BM + 9], 0.0)
        conv = a0 + accs[1][8:BM + 8] + a2
        t = jnp.maximum(conv + cb_ref[0, :][None, :], 0.0)
        h = jnp.dot(t.astype(_MM_DTYPE), hw_ref[...],
                    preferred_element_type=jnp.float32) + hb_ref[0, :][None, :]
        ht = jnp.transpose(h)
        oc_refs[li][n, :, m0:m0 + BM] = ht[:3]
        ob_refs[li][n, :, m0:m0 + BM] = ht[3:15]


@jax.jit
def _rpn_all(x3, x4, x5, x6, w3, cb, hw, hb):
    out_shape = []
    for S, _ in _LEVELS:
        out_shape += [jax.ShapeDtypeStruct((_N, 3, S * S), jnp.float32),
                      jax.ShapeDtypeStruct((_N, 12, S * S), jnp.float32)]

    any_spec = pl.BlockSpec(memory_space=pl.ANY)
    stages = [pltpu.VMEM((2, _C, _RB3, 128), jnp.float32)]
    for S, BM in _LEVELS[1:]:
        stages.append(pltpu.VMEM((2, BM + 2 * S + 16, _C), jnp.float32))

    o = pl.pallas_call(
        _rpn_body,
        in_specs=[any_spec] * 4 + [
            pl.BlockSpec((3, 3 * _C, _C), lambda: (0, 0, 0)),
            pl.BlockSpec((1, _C), lambda: (0, 0)),
            pl.BlockSpec((_C, _NH), lambda: (0, 0)),
            pl.BlockSpec((1, _NH), lambda: (0, 0)),
        ],
        out_shape=out_shape,
        scratch_shapes=stages + [
            pltpu.VMEM((_XTR, _C), _MM_DTYPE),
            pltpu.SemaphoreType.DMA((2,)),
        ],
    )(x3, x4, x5, x6, w3, cb, hw, hb)

    cls_out, bbox_out = [], []
    for li, (S, _) in enumerate(_LEVELS):
        cls_out.append(o[2 * li].reshape(_N, 3, S, S))
        bbox_out.append(o[2 * li + 1].reshape(_N, 12, S, S))
    return tuple(cls_out) + tuple(bbox_out)


def kernel(feat_p3, feat_p4, feat_p5, feat_p6,
           conv_w, conv_b, cls_w, cls_b, bbox_w, bbox_b):
    w3 = jnp.transpose(conv_w, (3, 2, 1, 0)).reshape(3, 3 * _C, _C)
    w3 = w3.astype(_MM_DTYPE)
    cb = conv_b.reshape(1, _C)
    hw = jnp.concatenate([cls_w[:, :, 0, 0], bbox_w[:, :, 0, 0]], axis=0)
    hw = jnp.pad(hw, ((0, 1), (0, 0))).T.astype(_MM_DTYPE)
    hb = jnp.pad(jnp.concatenate([cls_b, bbox_b]), (0, 1)).reshape(1, _NH)

    flat = [jnp.transpose(x, (0, 2, 3, 1)).reshape(_N, -1, _C)
            for x in (feat_p4, feat_p5, feat_p6)]
    return _rpn_all(feat_p3, *flat, w3, cb, hw, hb)


# speedup vs baseline: 1.5192x; 1.0355x over previous
import jax
import jax.numpy as jnp
from jax.experimental import pallas as pl
from jax.experimental.pallas import tpu as pltpu

_C = 256
_NH = 16
_MM_DTYPE = jnp.bfloat16
_N = 2
_LEVELS = ((128, 2048), (64, 2048), (32, 1024), (16, 256))


def _jobs():
    out = []
    for li, (S, BM) in enumerate(_LEVELS):
        SS = S * S
        for n in range(_N):
            for b in range(SS // BM):
                m0 = b * BM
                if li == 0:
                    r0, r1 = m0 // S, (m0 + BM) // S
                    lo = max(0, r0 - 2) // 8 * 8
                    hi = min(S, (min(S, r1 + 2) + 7) // 8 * 8)
                    out.append((li, n, S, BM, m0, lo, hi, 0))
                else:
                    lo = max(0, m0 - S - 8)
                    hi = min(SS, m0 + BM + S + 8)
                    out.append((li, n, S, BM, m0, lo, hi, lo - (m0 - S - 8)))
    return out


_RB3 = max(j[6] - j[5] for j in _jobs() if j[0] == 0)


def _rpn_body(x3_ref, x4_ref, x5_ref, x6_ref,
              w3p_ref, w3f_ref, cbp_ref, cbf_ref,
              hwp_ref, hwf_ref, hbp_ref, hbf_ref,
              oc3, ob3, oc4, ob4, oc5, ob5, oc6, ob6,
              st3, st4, st5, st6, sem_ref):
    x_refs = (x3_ref, x4_ref, x5_ref, x6_ref)
    st_refs = (st3, st4, st5, st6)
    oc_refs = (oc3, oc4, oc5, oc6)
    ob_refs = (ob3, ob4, ob5, ob6)
    jobs = _jobs()

    def copy(i):
        li, n, S, BM, m0, lo, hi, zhead = jobs[i]
        if li == 0:
            return pltpu.make_async_copy(
                x_refs[0].at[n, :, lo:hi, :],
                st_refs[0].at[i % 2, :, 0:hi - lo, :],
                sem_ref.at[i % 2])
        return pltpu.make_async_copy(
            x_refs[li].at[n, lo:hi, :],
            st_refs[li].at[i % 2, zhead:zhead + hi - lo, :],
            sem_ref.at[i % 2])

    copy(0).start()
    for i, (li, n, S, BM, m0, lo, hi, zhead) in enumerate(jobs):
        if i + 1 < len(jobs):
            copy(i + 1).start()
        buf = i % 2
        copy(i).wait()
        SS = S * S

        if li == 0:
            RB = hi - lo
            xval = st_refs[0][buf, :, 0:RB, :].reshape(_C, RB * S)
            xval = xval.astype(_MM_DTYPE)
            w_lo, w_hi = m0 - 2 * S, m0 + BM + 2 * S
            a, b2 = max(w_lo, 0) - lo * S, min(w_hi, SS) - lo * S
            parts = []
            if w_lo < 0:
                parts.append(jnp.zeros((_C, -w_lo), _MM_DTYPE))
            parts.append(xval[:, a:b2])
            if w_hi > SS:
                parts.append(jnp.zeros((_C, w_hi - SS), _MM_DTYPE))
            xwin = jnp.concatenate(parts, axis=1) if len(parts) > 1 else parts[0]
            xsh = (jnp.roll(xwin, 1, axis=1), xwin, jnp.roll(xwin, -1, axis=1))
            accs = []
            for dx in range(3):
                x3c = jnp.concatenate(
                    [xsh[dx][:, (dy + 1) * S:(dy + 1) * S + BM]
                     for dy in range(3)], axis=0)
                accs.append(jnp.dot(w3p_ref[dx], x3c,
                                    preferred_element_type=jnp.float32))
            col = (jax.lax.broadcasted_iota(jnp.int32, (1, BM), 1) + m0) & (S - 1)
            conv = (jnp.where(col != 0, accs[0], 0.0) + accs[1]
                    + jnp.where(col != S - 1, accs[2], 0.0))
            t = jnp.maximum(conv + cbp_ref[:, 0:1], 0.0)
            h = jnp.dot(hwp_ref[...], t.astype(_MM_DTYPE),
                        preferred_element_type=jnp.float32) + hbp_ref[:, 0:1]
            oc_refs[0][n, :, m0:m0 + BM] = h[:3]
            ob_refs[0][n, :, m0:m0 + BM] = h[3:15]
        else:
            rows = BM + 2 * S + 16
            if zhead:
                st_refs[li][buf, 0:zhead, :] = jnp.zeros(
                    (zhead, _C), jnp.float32)
            if zhead + hi - lo < rows:
                st_refs[li][buf, zhead + hi - lo:rows, :] = jnp.zeros(
                    (rows - zhead - hi + lo, _C), jnp.float32)
            xs3 = jnp.concatenate(
                [st_refs[li][buf, pl.ds(dy * S, BM + 16), :]
                 for dy in range(3)],
                axis=1).astype(_MM_DTYPE)
            accs = [jnp.dot(xs3, w3f_ref[dx],
                            preferred_element_type=jnp.float32)
                    for dx in range(3)]
            col = (jax.lax.broadcasted_iota(jnp.int32, (BM, 1), 0) + m0) & (S - 1)
            a0 = jnp.where(col != 0, accs[0][7:BM + 7], 0.0)
            a2 = jnp.where(col != S - 1, accs[2][9:BM + 9], 0.0)
            conv = a0 + accs[1][8:BM + 8] + a2
            t = jnp.maximum(conv + cbf_ref[0, :][None, :], 0.0)
            h = jnp.dot(t.astype(_MM_DTYPE), hwf_ref[...],
                        preferred_element_type=jnp.float32) + hbf_ref[0, :][None, :]
            ht = jnp.transpose(h)
            oc_refs[li][n, :, m0:m0 + BM] = ht[:3]
            ob_refs[li][n, :, m0:m0 + BM] = ht[3:15]


@jax.jit
def _rpn_all(x3, x4, x5, x6, w3p, w3f, cbp, cbf, hwp, hwf, hbp, hbf):
    out_shape = []
    for S, _ in _LEVELS:
        out_shape += [jax.ShapeDtypeStruct((_N, 3, S * S), jnp.float32),
                      jax.ShapeDtypeStruct((_N, 12, S * S), jnp.float32)]

    any_spec = pl.BlockSpec(memory_space=pl.ANY)
    stages = [pltpu.VMEM((2, _C, _RB3, 128), jnp.float32)]
    for S, BM in _LEVELS[1:]:
        stages.append(pltpu.VMEM((2, BM + 2 * S + 16, _C), jnp.float32))

    o = pl.pallas_call(
        _rpn_body,
        in_specs=[any_spec] * 4 + [
            pl.BlockSpec((3, _C, 3 * _C), lambda: (0, 0, 0)),
            pl.BlockSpec((3, 3 * _C, _C), lambda: (0, 0, 0)),
            pl.BlockSpec((_C, 1), lambda: (0, 0)),
            pl.BlockSpec((1, _C), lambda: (0, 0)),
            pl.BlockSpec((_NH, _C), lambda: (0, 0)),
            pl.BlockSpec((_C, _NH), lambda: (0, 0)),
            pl.BlockSpec((_NH, 1), lambda: (0, 0)),
            pl.BlockSpec((1, _NH), lambda: (0, 0)),
        ],
        out_shape=out_shape,
        scratch_shapes=stages + [pltpu.SemaphoreType.DMA((2,))],
    )(x3, x4, x5, x6, w3p, w3f, cbp, cbf, hwp, hwf, hbp, hbf)

    cls_out, bbox_out = [], []
    for li, (S, _) in enumerate(_LEVELS):
        cls_out.append(o[2 * li].reshape(_N, 3, S, S))
        bbox_out.append(o[2 * li + 1].reshape(_N, 12, S, S))
    return tuple(cls_out) + tuple(bbox_out)


def kernel(feat_p3, feat_p4, feat_p5, feat_p6,
           conv_w, conv_b, cls_w, cls_b, bbox_w, bbox_b):
    w3p = jnp.transpose(conv_w, (3, 0, 2, 1)).reshape(3, _C, 3 * _C)
    w3p = w3p.astype(_MM_DTYPE)
    w3f = jnp.transpose(conv_w, (3, 2, 1, 0)).reshape(3, 3 * _C, _C)
    w3f = w3f.astype(_MM_DTYPE)
    cbp = conv_b.reshape(_C, 1)
    cbf = conv_b.reshape(1, _C)
    hw0 = jnp.concatenate([cls_w[:, :, 0, 0], bbox_w[:, :, 0, 0]], axis=0)
    hwp = jnp.pad(hw0, ((0, 1), (0, 0))).astype(_MM_DTYPE)
    hwf = hwp.T
    hb0 = jnp.pad(jnp.concatenate([cls_b, bbox_b]), (0, 1))
    hbp = hb0.reshape(_NH, 1)
    hbf = hb0.reshape(1, _NH)

    flat = [jnp.transpose(x, (0, 2, 3, 1)).reshape(_N, -1, _C)
            for x in (feat_p4, feat_p5, feat_p6)]
    return _rpn_all(feat_p3, *flat, w3p, w3f, cbp, cbf, hwp, hwf, hbp, hbf)
